# baseline (device time: 335014 ns/iter reference)
import jax
import jax.numpy as jnp
from jax import lax
from jax.experimental import pallas as pl
from jax.experimental.pallas import tpu as pltpu

N_DEV = 16
ROUNDS = N_DEV // 2 - 1
WIRE_DTYPE = jnp.float8_e4m3fn


def kernel(x, router_W, route_idx, expert_W, shared_W):
    n_tok, d_model = x.shape
    e_per, _, d_ff = expert_W.shape
    n_exp = N_DEV * e_per
    k_cat = e_per * d_model

    def body(x_ref, rw_ref, idx_ref, ew_ref, sw_ref, out_ref,
             commR_ref, commL_ref, pbuf_ref, psrc_ref, xcat_ref,
             send_semsR, recv_semsR, send_semsL, recv_semsL,
             p_send_sem, p_recv_sem, readyR, readyL):
        my = lax.axis_index("i")
        left = lax.rem(my - 1 + N_DEV, N_DEV)
        right = lax.rem(my + 1, N_DEV)
        partner = lax.rem(my + N_DEV // 2, N_DEV)

        barrier_sem = pltpu.get_barrier_semaphore()
        for nbr in (left, right, partner):
            pl.semaphore_signal(barrier_sem, inc=1, device_id=(nbr,),
                                device_id_type=pl.DeviceIdType.MESH)
        pl.semaphore_wait(barrier_sem, 3)

        w_own = ew_ref[:, :, :].astype(WIRE_DTYPE).reshape(k_cat, d_ff)
        commR_ref[0, :, :] = w_own
        commL_ref[0, :, :] = w_own
        psrc_ref[:, :] = w_own

        state = {}

        def local_prep():
            x_bf = x_ref[:, :].astype(jnp.bfloat16)
            scores = jnp.dot(x_bf, rw_ref[:, :].astype(jnp.bfloat16),
                             preferred_element_type=jnp.float32)
            m = jnp.max(scores, axis=1, keepdims=True)
            ex = jnp.exp(scores - m)
            probs = ex / jnp.sum(ex, axis=1, keepdims=True)
            idx = idx_ref[:, :]
            e_iota = lax.broadcasted_iota(jnp.int32, (n_tok, n_exp), 1)
            p = jnp.sum(jnp.where(e_iota == idx, probs, 0.0), axis=1,
                        keepdims=True)
            out_ref[:, :] = jnp.dot(x_bf, sw_ref[:, :].astype(jnp.bfloat16),
                                    preferred_element_type=jnp.float32)
            state["x_bf"] = x_bf
            state["idx"] = idx
            state["p"] = p

        def block_compute(ref, idx2d, owner):
            for j in range(e_per):
                e_g = owner * e_per + j
                coeff = jnp.where(state["idx"] == e_g, state["p"],
                                  0.0).astype(jnp.bfloat16)
                xcat_ref[:, j * d_model:(j + 1) * d_model] = (
                    state["x_bf"] * coeff)
            w = (ref[idx2d, :, :] if idx2d is not None
                 else ref[:, :]).astype(jnp.bfloat16)
            out_ref[:, :] += jnp.dot(xcat_ref[:, :], w,
                                     preferred_element_type=jnp.float32)

        p_recv = pltpu.make_async_remote_copy(
            src_ref=pbuf_ref, dst_ref=pbuf_ref,
            send_sem=p_send_sem.at[0], recv_sem=p_recv_sem.at[0],
            device_id=(partner,), device_id_type=pl.DeviceIdType.MESH,
        )

        for r in range(ROUNDS):
            s = r % 2
            d = (r + 1) % 2
            if r >= 2:
                pl.semaphore_wait(readyR, 1)
            rdmaR = pltpu.make_async_remote_copy(
                src_ref=commR_ref.at[s], dst_ref=commR_ref.at[d],
                send_sem=send_semsR.at[s], recv_sem=recv_semsR.at[d],
                device_id=(right,), device_id_type=pl.DeviceIdType.MESH,
            )
            rdmaR.start()
            if r >= 2:
                pl.semaphore_wait(readyL, 1)
            rdmaL = pltpu.make_async_remote_copy(
                src_ref=commL_ref.at[s], dst_ref=commL_ref.at[d],
                send_sem=send_semsL.at[s], recv_sem=recv_semsL.at[d],
                device_id=(left,), device_id_type=pl.DeviceIdType.MESH,
            )
            rdmaL.start()
            if r == 0:
                p_send = pltpu.make_async_remote_copy(
                    src_ref=psrc_ref, dst_ref=pbuf_ref,
                    send_sem=p_send_sem.at[0], recv_sem=p_recv_sem.at[0],
                    device_id=(partner,), device_id_type=pl.DeviceIdType.MESH,
                )
                p_send.start()

            if r == 0:
                local_prep()
                block_compute(commR_ref, 0, my)
            else:
                block_compute(commR_ref, s, lax.rem(my - r + N_DEV, N_DEV))
                block_compute(commL_ref, s, lax.rem(my + r, N_DEV))
            rdmaR.wait_send()
            if r == 0:
                p_send.wait_send()
            if 1 <= r <= ROUNDS - 2:
                pl.semaphore_signal(readyR, inc=1, device_id=(left,),
                                    device_id_type=pl.DeviceIdType.MESH)
            rdmaL.wait_send()
            if 1 <= r <= ROUNDS - 2:
                pl.semaphore_signal(readyL, inc=1, device_id=(right,),
                                    device_id_type=pl.DeviceIdType.MESH)
            rdmaR.wait_recv()
            rdmaL.wait_recv()

        block_compute(commR_ref, ROUNDS % 2, lax.rem(my - ROUNDS + N_DEV, N_DEV))
        block_compute(commL_ref, ROUNDS % 2, lax.rem(my + ROUNDS, N_DEV))
        p_recv.wait_recv()
        block_compute(pbuf_ref, None, lax.rem(my + N_DEV // 2, N_DEV))

    return pl.pallas_call(
        body,
        out_shape=jax.ShapeDtypeStruct((n_tok, d_ff), jnp.float32),
        in_specs=[pl.BlockSpec(memory_space=pltpu.VMEM)] * 5,
        out_specs=pl.BlockSpec(memory_space=pltpu.VMEM),
        scratch_shapes=[
            pltpu.VMEM((2, k_cat, d_ff), WIRE_DTYPE),
            pltpu.VMEM((2, k_cat, d_ff), WIRE_DTYPE),
            pltpu.VMEM((k_cat, d_ff), WIRE_DTYPE),
            pltpu.VMEM((k_cat, d_ff), WIRE_DTYPE),
            pltpu.VMEM((n_tok, k_cat), jnp.bfloat16),
            pltpu.SemaphoreType.DMA((2,)),
            pltpu.SemaphoreType.DMA((2,)),
            pltpu.SemaphoreType.DMA((2,)),
            pltpu.SemaphoreType.DMA((2,)),
            pltpu.SemaphoreType.DMA((1,)),
            pltpu.SemaphoreType.DMA((1,)),
            pltpu.SemaphoreType.REGULAR,
            pltpu.SemaphoreType.REGULAR,
        ],
        compiler_params=pltpu.CompilerParams(collective_id=0),
    )(x, router_W, route_idx, expert_W, shared_W)


# device time: 214327 ns/iter; 1.5631x vs baseline; 1.5631x over previous
import jax
import jax.numpy as jnp
from jax import lax
from jax.experimental import pallas as pl
from jax.experimental.pallas import tpu as pltpu

N_DEV = 16
ROUNDS = N_DEV // 2
WIRE_DTYPE = jnp.float8_e4m3fn


def kernel(x, router_W, route_idx, expert_W, shared_W):
    n_tok, d_model = x.shape
    e_per, _, d_ff = expert_W.shape
    n_exp = N_DEV * e_per
    k_cat = e_per * d_model
    k_half = k_cat // 2

    def body(x_ref, rw_ref, idx_ref, ew_ref, sw_ref, out_ref,
             commR_ref, commL_ref, xcat_ref,
             send_semsR, recv_semsR, send_semsL, recv_semsL,
             readyR, readyL):
        my = lax.axis_index("i")
        left = lax.rem(my - 1 + N_DEV, N_DEV)
        right = lax.rem(my + 1, N_DEV)

        barrier_sem = pltpu.get_barrier_semaphore()
        for nbr in (left, right):
            pl.semaphore_signal(barrier_sem, inc=1, device_id=(nbr,),
                                device_id_type=pl.DeviceIdType.MESH)
        pl.semaphore_wait(barrier_sem, 2)

        w_own = ew_ref[:, :, :].astype(WIRE_DTYPE).reshape(k_cat, d_ff)
        commR_ref[0, :, :] = w_own
        commL_ref[0, :, :] = w_own

        state = {}

        def local_prep():
            x_bf = x_ref[:, :].astype(jnp.bfloat16)
            scores = jnp.dot(x_bf, rw_ref[:, :].astype(jnp.bfloat16),
                             preferred_element_type=jnp.float32)
            m = jnp.max(scores, axis=1, keepdims=True)
            ex = jnp.exp(scores - m)
            probs = ex / jnp.sum(ex, axis=1, keepdims=True)
            idx = idx_ref[:, :]
            e_iota = lax.broadcasted_iota(jnp.int32, (n_tok, n_exp), 1)
            p = jnp.sum(jnp.where(e_iota == idx, probs, 0.0), axis=1,
                        keepdims=True)
            out_ref[:, :] = jnp.dot(x_bf, sw_ref[:, :].astype(jnp.bfloat16),
                                    preferred_element_type=jnp.float32)
            state["x_bf"] = x_bf
            state["idx"] = idx
            state["p"] = p

        def build_xcat(owner):
            for j in range(e_per):
                e_g = owner * e_per + j
                coeff = jnp.where(state["idx"] == e_g, state["p"],
                                  0.0).astype(jnp.bfloat16)
                xcat_ref[:, j * d_model:(j + 1) * d_model] = (
                    state["x_bf"] * coeff)

        def block_compute(comm_ref, slot, owner):
            build_xcat(owner)
            w = comm_ref[slot, :, :].astype(jnp.bfloat16)
            out_ref[:, :] += jnp.dot(xcat_ref[:, :], w,
                                     preferred_element_type=jnp.float32)

        for r in range(ROUNDS):
            s = r % 2
            d = (r + 1) % 2
            last = r == ROUNDS - 1
            if r >= 2:
                pl.semaphore_wait(readyR, 1)
            rdmaR = pltpu.make_async_remote_copy(
                src_ref=(commR_ref.at[s, 0:k_half] if last
                         else commR_ref.at[s]),
                dst_ref=(commR_ref.at[d, 0:k_half] if last
                         else commR_ref.at[d]),
                send_sem=send_semsR.at[s], recv_sem=recv_semsR.at[d],
                device_id=(right,), device_id_type=pl.DeviceIdType.MESH,
            )
            rdmaR.start()
            if r >= 2:
                pl.semaphore_wait(readyL, 1)
            rdmaL = pltpu.make_async_remote_copy(
                src_ref=(commL_ref.at[s, k_half:k_cat] if last
                         else commL_ref.at[s]),
                dst_ref=(commL_ref.at[d, k_half:k_cat] if last
                         else commL_ref.at[d]),
                send_sem=send_semsL.at[s], recv_sem=recv_semsL.at[d],
                device_id=(left,), device_id_type=pl.DeviceIdType.MESH,
            )
            rdmaL.start()

            if r == 0:
                local_prep()
                block_compute(commR_ref, 0, my)
            else:
                block_compute(commR_ref, s, lax.rem(my - r + N_DEV, N_DEV))
                block_compute(commL_ref, s, lax.rem(my + r, N_DEV))

            rdmaR.wait_send()
            if 1 <= r <= ROUNDS - 2:
                pl.semaphore_signal(readyR, inc=1, device_id=(left,),
                                    device_id_type=pl.DeviceIdType.MESH)
            rdmaL.wait_send()
            if 1 <= r <= ROUNDS - 2:
                pl.semaphore_signal(readyL, inc=1, device_id=(right,),
                                    device_id_type=pl.DeviceIdType.MESH)
            rdmaR.wait_recv()
            rdmaL.wait_recv()

        build_xcat(lax.rem(my + N_DEV // 2, N_DEV))
        w_anti = jnp.concatenate(
            [commR_ref[0, 0:k_half, :], commL_ref[0, k_half:k_cat, :]],
            axis=0).astype(jnp.bfloat16)
        out_ref[:, :] += jnp.dot(xcat_ref[:, :], w_anti,
                                 preferred_element_type=jnp.float32)

    return pl.pallas_call(
        body,
        out_shape=jax.ShapeDtypeStruct((n_tok, d_ff), jnp.float32),
        in_specs=[pl.BlockSpec(memory_space=pltpu.VMEM)] * 5,
        out_specs=pl.BlockSpec(memory_space=pltpu.VMEM),
        scratch_shapes=[
            pltpu.VMEM((2, k_cat, d_ff), WIRE_DTYPE),
            pltpu.VMEM((2, k_cat, d_ff), WIRE_DTYPE),
            pltpu.VMEM((n_tok, k_cat), jnp.bfloat16),
            pltpu.SemaphoreType.DMA((2,)),
            pltpu.SemaphoreType.DMA((2,)),
            pltpu.SemaphoreType.DMA((2,)),
            pltpu.SemaphoreType.DMA((2,)),
            pltpu.SemaphoreType.REGULAR,
            pltpu.SemaphoreType.REGULAR,
        ],
        compiler_params=pltpu.CompilerParams(collective_id=0),
    )(x, router_W, route_idx, expert_W, shared_W)
